# Initial kernel scaffold; baseline (speedup 1.0000x reference)
import jax
import jax.numpy as jnp
from jax.experimental import pallas as pl


def kernel(gallery_feat, query_feats, gallery_label, query_labels):
    raise NotImplementedError("write your pallas kernel here")


# speedup vs baseline: 3.2266x; 3.2266x over previous
import jax
import jax.numpy as jnp
from jax.experimental import pallas as pl
from jax.experimental.pallas import tpu as pltpu

_TOP_K = 50


def _matvec_kernel(q_ref, w_ref, o_ref):
    o_ref[...] = jax.lax.dot_general(
        q_ref[...], w_ref[...], (((1,), (0,)), ((), ())),
        preferred_element_type=jnp.float32,
    )


def _loss_kernel(sim_ref, lab_ref, gl_ref, o_ref, s_ref):
    sim = sim_ref[...]
    mask = lab_ref[...] == gl_ref[0, 0]
    pos_cnt = jnp.sum(mask.astype(jnp.float32))
    pos_sum = jnp.sum(jnp.where(mask, sim, 0.0))
    pos_max = jnp.max(jnp.where(mask, sim, -jnp.inf))
    neg = jnp.where(mask, -jnp.inf, sim)
    neg_max = jnp.max(neg)
    m_all = jnp.maximum(pos_max, neg_max)
    pos_es = jnp.sum(jnp.where(mask, jnp.exp(sim - m_all), 0.0))

    s_ref[...] = neg

    def body(_, carry):
        taken, es = carry
        cur = s_ref[...]
        m = jnp.max(cur)
        eqm = cur == m
        cnt = jnp.sum(eqm.astype(jnp.float32))
        take = jnp.clip(jnp.minimum(cnt, _TOP_K - taken), 0.0, None)
        s_ref[...] = jnp.where(eqm, -jnp.inf, cur)
        return taken + cnt, es + take * jnp.exp(m - m_all)

    _, neg_es = jax.lax.fori_loop(0, _TOP_K, body, (0.0, 0.0))

    exp_sum = pos_es + neg_es
    lossv = -(pos_sum / jnp.maximum(pos_cnt, 1.0)) + m_all + jnp.log(exp_sum)
    o_ref[...] = jnp.where(pos_cnt == 0.0, 0.0, lossv)[None, None]


def kernel(gallery_feat, query_feats, gallery_label, query_labels):
    n, d = query_feats.shape
    q = 8
    r = n // q
    q2 = query_feats.reshape(r, q * d)

    eye = jnp.eye(q, dtype=jnp.float32)
    w = (eye[:, None, :] * gallery_feat[None, :, None]).reshape(q * d, q)

    br = 5000
    nb = r // br
    sim = pl.pallas_call(
        _matvec_kernel,
        grid=(nb,),
        in_specs=[
            pl.BlockSpec((br, q * d), lambda i: (i, 0)),
            pl.BlockSpec((q * d, q), lambda i: (0, 0)),
        ],
        out_specs=pl.BlockSpec((br, q), lambda i: (i, 0)),
        out_shape=jax.ShapeDtypeStruct((r, q), jnp.float32),
    )(q2, w)

    rows, cols = 125, 8000
    sim2 = sim.reshape(rows, cols)
    lab2 = query_labels.reshape(rows, cols)
    gl = gallery_label.reshape(1, 1)

    loss = pl.pallas_call(
        _loss_kernel,
        out_shape=jax.ShapeDtypeStruct((1, 1), jnp.float32),
        scratch_shapes=[pltpu.VMEM((rows, cols), jnp.float32)],
    )(sim2, lab2, gl)
    return loss[0, 0]
